# baseline (speedup 1.0000x reference)
import functools

import jax
import jax.numpy as jnp
from jax import lax
from jax.experimental import pallas as pl
from jax.experimental.pallas import tpu as pltpu
from jax.experimental.pallas import tpu_sc as plsc

B = 1024
L = 200
H = 64
V = 1000000
NC = 2
NS = 16
NW = NC * NS
LANES = 16

TBLK = 128
NFULL = V // TBLK
TAIL = V - NFULL * TBLK
BLK_PER_W = -(-NFULL // NW)

BBLK = 128
NBBLK = B // BBLK
L_PER_W = L * NBBLK // NW
UNITS = L_PER_W


def _iota16():
    return lax.iota(jnp.int32, LANES)


def _transpose_block(src_v, stg_v, ntok):
    rvecs = [k * LANES + _iota16() for k in range(H // LANES)]

    def tbody(t, carry):
        col = jnp.full((LANES,), 0, jnp.int32) + t
        for k in range(H // LANES):
            vec = plsc.load_gather(src_v, [rvecs[k], col])
            stg_v[t, pl.ds(k * LANES, LANES)] = vec
        return carry

    lax.fori_loop(0, ntok, tbody, 0, unroll=4)


def _tr_body(tokT_hbm, tailT_hbm, out_hbm, in0, in1, stg, tail_v, sem0, sem1):
    w = lax.axis_index("s") * NC + lax.axis_index("c")

    first = w
    pltpu.async_copy(tokT_hbm.at[:, pl.ds(first * TBLK, TBLK)], in0, sem0)

    def pair(j, carry):
        blk0 = w + NW * (2 * j)
        blk1 = w + NW * (2 * j + 1)

        @pl.when(blk1 < NFULL)
        def _():
            pltpu.async_copy(tokT_hbm.at[:, pl.ds(blk1 * TBLK, TBLK)], in1, sem1)

        @pl.when(blk0 < NFULL)
        def _():
            pltpu.make_async_copy(
                tokT_hbm.at[:, pl.ds(blk0 * TBLK, TBLK)], in0, sem0
            ).wait()
            _transpose_block(in0, stg, TBLK)
            pltpu.sync_copy(stg, out_hbm.at[pl.ds(blk0 * TBLK, TBLK)])
            blk2 = blk0 + 2 * NW

            @pl.when(blk2 < NFULL)
            def _():
                pltpu.async_copy(
                    tokT_hbm.at[:, pl.ds(blk2 * TBLK, TBLK)], in0, sem0
                )

        @pl.when(blk1 < NFULL)
        def _():
            pltpu.make_async_copy(
                tokT_hbm.at[:, pl.ds(blk1 * TBLK, TBLK)], in1, sem1
            ).wait()
            _transpose_block(in1, stg, TBLK)
            pltpu.sync_copy(stg, out_hbm.at[pl.ds(blk1 * TBLK, TBLK)])

        return carry

    lax.fori_loop(0, (BLK_PER_W + 1) // 2, pair, 0)

    @pl.when(w == NW - 1)
    def _():
        pltpu.sync_copy(tailT_hbm, tail_v)

        def tb(t, carry):
            for k in range(H // LANES):
                s = pl.ds(k * LANES, LANES)
                stg[t, s] = tail_v[t, s]
            return carry

        lax.fori_loop(0, TAIL, tb, 0)
        pltpu.sync_copy(
            stg.at[pl.ds(0, TAIL)], out_hbm.at[pl.ds(NFULL * TBLK, TAIL)]
        )


def _add_write(out_hbm, pos_v, rows_v, stg_v, l0, bblk, u):
    p = [pos_v[u, pl.ds(k * LANES, LANES)] for k in range(H // LANES)]

    def rbody(r, carry):
        for k in range(H // LANES):
            s = pl.ds(k * LANES, LANES)
            stg_v[r, s] = rows_v[r, s] + p[k]
        return carry

    lax.fori_loop(0, BBLK, rbody, 0, unroll=4)
    pltpu.sync_copy(
        stg_v,
        out_hbm.at[pl.ds(bblk * BBLK, BBLK), pl.ds((l0 + u) * H, H)],
    )


def _gt_body(xT_hbm, tok_hbm, pos_hbm, out_hbm,
             idx_v, pos_v, rows0, rows1, stg_v, sem0, sem1):
    w = lax.axis_index("s") * NC + lax.axis_index("c")
    bblk = lax.rem(w, NBBLK)
    l0 = lax.div(w, NBBLK) * L_PER_W
    pltpu.sync_copy(
        xT_hbm.at[pl.ds(l0, L_PER_W), pl.ds(bblk * BBLK, BBLK)], idx_v
    )
    pltpu.sync_copy(pos_hbm.at[pl.ds(l0, L_PER_W)], pos_v)

    pltpu.async_copy(tok_hbm.at[idx_v.at[0]], rows0, sem0)

    def pair(j, carry):
        a = 2 * j
        b = a + 1
        pltpu.async_copy(tok_hbm.at[idx_v.at[b]], rows1, sem1)
        pltpu.make_async_copy(tok_hbm.at[idx_v.at[a]], rows0, sem0).wait()
        _add_write(out_hbm, pos_v, rows0, stg_v, l0, bblk, a)
        nxt = jnp.minimum(a + 2, UNITS - 1)
        pltpu.async_copy(tok_hbm.at[idx_v.at[nxt]], rows0, sem0)
        pltpu.make_async_copy(tok_hbm.at[idx_v.at[b]], rows1, sem1).wait()
        _add_write(out_hbm, pos_v, rows1, stg_v, l0, bblk, b)
        return carry

    lax.fori_loop(0, UNITS // 2, pair, 0)
    pltpu.make_async_copy(tok_hbm.at[idx_v.at[0]], rows0, sem0).wait()


@functools.cache
def _build_transpose():
    return functools.partial(
        pl.kernel,
        out_type=jax.ShapeDtypeStruct((V, 2 * H), jnp.float32),
        mesh=plsc.VectorSubcoreMesh(core_axis_name="c", subcore_axis_name="s"),
        scratch_types=[
            pltpu.VMEM((H, TBLK), jnp.float32),
            pltpu.VMEM((H, TBLK), jnp.float32),
            pltpu.VMEM((TBLK, 2 * H), jnp.float32),
            pltpu.VMEM((TAIL, H), jnp.float32),
            pltpu.SemaphoreType.DMA,
            pltpu.SemaphoreType.DMA,
        ],
        compiler_params=pltpu.CompilerParams(needs_layout_passes=False),
    )(_tr_body)


@functools.cache
def _build_gather():
    return functools.partial(
        pl.kernel,
        out_type=jax.ShapeDtypeStruct((B, L * H), jnp.float32),
        mesh=plsc.VectorSubcoreMesh(core_axis_name="c", subcore_axis_name="s"),
        scratch_types=[
            pltpu.VMEM((L_PER_W, BBLK), jnp.int32),
            pltpu.VMEM((L_PER_W, H), jnp.float32),
            pltpu.VMEM((BBLK, 2 * H), jnp.float32),
            pltpu.VMEM((BBLK, 2 * H), jnp.float32),
            pltpu.VMEM((BBLK, H), jnp.float32),
            pltpu.SemaphoreType.DMA,
            pltpu.SemaphoreType.DMA,
        ],
        compiler_params=pltpu.CompilerParams(use_tc_tiling_on_sc=False),
    )(_gt_body)


def kernel(x, token_table, pos_table):
    tailT = token_table[NFULL * TBLK:, :]
    tok_rows = _build_transpose()(token_table.T, tailT)
    out = _build_gather()(x.T, tok_rows, pos_table)
    return out.reshape(B, L, H)


# speedup vs baseline: 1.0482x; 1.0482x over previous
import functools

import jax
import jax.numpy as jnp
from jax import lax
from jax.experimental import pallas as pl
from jax.experimental.pallas import tpu as pltpu
from jax.experimental.pallas import tpu_sc as plsc

B = 1024
L = 200
H = 64
V = 1000000
VP = V // 2
NC = 2
NS = 16
NW = NC * NS
LANES = 16

TBLK = 128
PBLK = TBLK // 2
NFULL = V // TBLK
TAIL = V - NFULL * TBLK
BLK_PER_W = -(-NFULL // NW)
APAIRS = -(-BLK_PER_W // 2)

BBLK = 128
NBBLK = B // BBLK
L_PER_W = L * NBBLK // NW
UNITS = L_PER_W


def _i16():
    return lax.iota(jnp.int32, LANES)


def _pack_block(src_v, stg_v, ngrp):
    def tgroup(g, carry):
        t0 = g * LANES
        t = t0 + _i16()
        rows = t >> 1
        colb = (t & 1) * H
        for f in range(H):
            vec = src_v[f, pl.ds(t0, LANES)]
            plsc.store_scatter(stg_v, [rows, colb + f], vec)
        return carry

    lax.fori_loop(0, ngrp, tgroup, 0)


def _tr_body(tokT_hbm, tailT_hbm, out_hbm,
             in0, in1, stg0, stg1, tail_v, rs0, rs1, ws0, ws1):
    w = lax.axis_index("s") * NC + lax.axis_index("c")

    pltpu.async_copy(tokT_hbm.at[:, pl.ds(w * TBLK, TBLK)], in0, rs0)

    def drain_w(stg, sem):
        pltpu.make_async_copy(stg, out_hbm.at[pl.ds(0, PBLK)], sem).wait()

    def pair(j, carry):
        blk0 = w + NW * (2 * j)
        blk1 = w + NW * (2 * j + 1)

        @pl.when(blk1 < NFULL)
        def _():
            pltpu.async_copy(tokT_hbm.at[:, pl.ds(blk1 * TBLK, TBLK)], in1, rs1)

        @pl.when(blk0 < NFULL)
        def _():
            pltpu.make_async_copy(
                tokT_hbm.at[:, pl.ds(blk0 * TBLK, TBLK)], in0, rs0
            ).wait()

            @pl.when(j > 0)
            def _():
                drain_w(stg0, ws0)

            _pack_block(in0, stg0, TBLK // LANES)
            pltpu.async_copy(stg0, out_hbm.at[pl.ds(blk0 * PBLK, PBLK)], ws0)
            blk2 = blk0 + 2 * NW

            @pl.when(blk2 < NFULL)
            def _():
                pltpu.async_copy(
                    tokT_hbm.at[:, pl.ds(blk2 * TBLK, TBLK)], in0, rs0
                )

        @pl.when(blk1 < NFULL)
        def _():
            pltpu.make_async_copy(
                tokT_hbm.at[:, pl.ds(blk1 * TBLK, TBLK)], in1, rs1
            ).wait()

            @pl.when(j > 0)
            def _():
                drain_w(stg1, ws1)

            _pack_block(in1, stg1, TBLK // LANES)
            pltpu.async_copy(stg1, out_hbm.at[pl.ds(blk1 * PBLK, PBLK)], ws1)

        return carry

    lax.fori_loop(0, APAIRS, pair, 0)
    drain_w(stg0, ws0)
    drain_w(stg1, ws1)

    @pl.when(w == NW - 1)
    def _():
        pltpu.sync_copy(tailT_hbm, tail_v)

        def tb(t, carry):
            r = t >> 1
            cb = (t & 1) * H
            for k in range(H // LANES):
                stg0[r, pl.ds(cb + k * LANES, LANES)] = (
                    tail_v[t, pl.ds(k * LANES, LANES)]
                )
            return carry

        lax.fori_loop(0, TAIL, tb, 0)
        pltpu.sync_copy(
            stg0.at[pl.ds(0, TAIL // 2)],
            out_hbm.at[pl.ds(NFULL * PBLK, TAIL // 2)],
        )


def _compact_add(rows_v, par_v, pos_v, stg, u):
    def grp(g, carry):
        t0 = g * LANES
        tvec = t0 + _i16()
        colb = par_v[u, pl.ds(t0, LANES)] * H
        for f in range(H):
            vec = plsc.load_gather(rows_v, [tvec, colb + f])
            plsc.store_scatter(stg, [tvec, jnp.full((LANES,), f, jnp.int32)], vec)
        return carry

    lax.fori_loop(0, BBLK // LANES, grp, 0)

    p = [pos_v[u, pl.ds(k * LANES, LANES)] for k in range(H // LANES)]

    def rbody(r, carry):
        for k in range(H // LANES):
            s = pl.ds(k * LANES, LANES)
            stg[r, s] = stg[r, s] + p[k]
        return carry

    lax.fori_loop(0, BBLK, rbody, 0, unroll=4)


def _gt_body(xpT_hbm, xrT_hbm, tok_hbm, pos_hbm, out_hbm,
             idx_v, par_v, pos_v, rows0, rows1, stg0, stg1,
             gs0, gs1, ws0, ws1):
    w = lax.axis_index("s") * NC + lax.axis_index("c")
    bblk = lax.rem(w, NBBLK)
    l0 = lax.div(w, NBBLK) * L_PER_W
    cols = pl.ds(bblk * BBLK, BBLK)
    pltpu.sync_copy(xpT_hbm.at[pl.ds(l0, L_PER_W), cols], idx_v)
    pltpu.sync_copy(xrT_hbm.at[pl.ds(l0, L_PER_W), cols], par_v)
    pltpu.sync_copy(pos_hbm.at[pl.ds(l0, L_PER_W)], pos_v)

    def out_slice(u):
        return out_hbm.at[pl.ds(bblk * BBLK, BBLK), pl.ds((l0 + u) * H, H)]

    def drain_w(stg, sem):
        pltpu.make_async_copy(stg, out_slice(0), sem).wait()

    pltpu.async_copy(tok_hbm.at[idx_v.at[0]], rows0, gs0)

    def pair(j, carry):
        a = 2 * j
        b = a + 1
        pltpu.async_copy(tok_hbm.at[idx_v.at[b]], rows1, gs1)
        pltpu.make_async_copy(tok_hbm.at[idx_v.at[a]], rows0, gs0).wait()

        @pl.when(j > 0)
        def _():
            drain_w(stg0, ws0)

        _compact_add(rows0, par_v, pos_v, stg0, a)
        pltpu.async_copy(stg0, out_slice(a), ws0)
        nxt = jnp.minimum(a + 2, UNITS - 1)
        pltpu.async_copy(tok_hbm.at[idx_v.at[nxt]], rows0, gs0)
        pltpu.make_async_copy(tok_hbm.at[idx_v.at[b]], rows1, gs1).wait()

        @pl.when(j > 0)
        def _():
            drain_w(stg1, ws1)

        _compact_add(rows1, par_v, pos_v, stg1, b)
        pltpu.async_copy(stg1, out_slice(b), ws1)
        return carry

    lax.fori_loop(0, UNITS // 2, pair, 0)
    pltpu.make_async_copy(tok_hbm.at[idx_v.at[0]], rows0, gs0).wait()
    drain_w(stg0, ws0)
    drain_w(stg1, ws1)


@functools.cache
def _build_transpose():
    return functools.partial(
        pl.kernel,
        out_type=jax.ShapeDtypeStruct((VP, 2 * H), jnp.float32),
        mesh=plsc.VectorSubcoreMesh(core_axis_name="c", subcore_axis_name="s"),
        scratch_types=[
            pltpu.VMEM((H, TBLK), jnp.float32),
            pltpu.VMEM((H, TBLK), jnp.float32),
            pltpu.VMEM((PBLK, 2 * H), jnp.float32),
            pltpu.VMEM((PBLK, 2 * H), jnp.float32),
            pltpu.VMEM((TAIL, H), jnp.float32),
            pltpu.SemaphoreType.DMA,
            pltpu.SemaphoreType.DMA,
            pltpu.SemaphoreType.DMA,
            pltpu.SemaphoreType.DMA,
        ],
        compiler_params=pltpu.CompilerParams(needs_layout_passes=False),
    )(_tr_body)


@functools.cache
def _build_gather():
    return functools.partial(
        pl.kernel,
        out_type=jax.ShapeDtypeStruct((B, L * H), jnp.float32),
        mesh=plsc.VectorSubcoreMesh(core_axis_name="c", subcore_axis_name="s"),
        scratch_types=[
            pltpu.VMEM((L_PER_W, BBLK), jnp.int32),
            pltpu.VMEM((L_PER_W, BBLK), jnp.int32),
            pltpu.VMEM((L_PER_W, H), jnp.float32),
            pltpu.VMEM((BBLK, 2 * H), jnp.float32),
            pltpu.VMEM((BBLK, 2 * H), jnp.float32),
            pltpu.VMEM((BBLK, H), jnp.float32),
            pltpu.VMEM((BBLK, H), jnp.float32),
            pltpu.SemaphoreType.DMA,
            pltpu.SemaphoreType.DMA,
            pltpu.SemaphoreType.DMA,
            pltpu.SemaphoreType.DMA,
        ],
        compiler_params=pltpu.CompilerParams(
            use_tc_tiling_on_sc=False, needs_layout_passes=False
        ),
    )(_gt_body)


def kernel(x, token_table, pos_table):
    tailT = token_table[NFULL * TBLK:, :]
    tok_rows = _build_transpose()(token_table.T, tailT)
    xp = (x >> 1).T
    xr = (x & 1).T
    out = _build_gather()(xp, xr, tok_rows, pos_table)
    return out.reshape(B, L, H)


# speedup vs baseline: 1.0488x; 1.0006x over previous
import functools

import jax
import jax.numpy as jnp
from jax import lax
from jax.experimental import pallas as pl
from jax.experimental.pallas import tpu as pltpu
from jax.experimental.pallas import tpu_sc as plsc

B = 1024
L = 200
H = 64
V = 1000000
VP = V // 2
NC = 2
NS = 16
NW = NC * NS
LANES = 16

TBLK = 128
PBLK = TBLK // 2
NFULL = V // TBLK
TAIL = V - NFULL * TBLK
BLK_PER_W = -(-NFULL // NW)
APAIRS = -(-BLK_PER_W // 2)

BBLK = 128
NBBLK = B // BBLK
L_PER_W = L * NBBLK // NW
UNITS = L_PER_W


def _i16():
    return lax.iota(jnp.int32, LANES)


def _pack_block(src_v, stg_v, ngrp):
    def tgroup(g, carry):
        t0 = g * LANES
        t = t0 + _i16()
        rows = t >> 1
        colb = (t & 1) * H
        for f in range(H):
            vec = src_v[f, pl.ds(t0, LANES)]
            plsc.store_scatter(stg_v, [rows, colb + f], vec)
        return carry

    lax.fori_loop(0, ngrp, tgroup, 0)


def _tr_body(tokT_hbm, tailT_hbm, out_hbm,
             in0, in1, stg0, stg1, tail_v, rs0, rs1, ws0, ws1):
    w = lax.axis_index("s") * NC + lax.axis_index("c")

    pltpu.async_copy(tokT_hbm.at[:, pl.ds(w * TBLK, TBLK)], in0, rs0)

    def drain_w(stg, sem):
        pltpu.make_async_copy(stg, out_hbm.at[pl.ds(0, PBLK)], sem).wait()

    def pair(j, carry):
        blk0 = w + NW * (2 * j)
        blk1 = w + NW * (2 * j + 1)

        @pl.when(blk1 < NFULL)
        def _():
            pltpu.async_copy(tokT_hbm.at[:, pl.ds(blk1 * TBLK, TBLK)], in1, rs1)

        @pl.when(blk0 < NFULL)
        def _():
            pltpu.make_async_copy(
                tokT_hbm.at[:, pl.ds(blk0 * TBLK, TBLK)], in0, rs0
            ).wait()

            @pl.when(j > 0)
            def _():
                drain_w(stg0, ws0)

            _pack_block(in0, stg0, TBLK // LANES)
            pltpu.async_copy(stg0, out_hbm.at[pl.ds(blk0 * PBLK, PBLK)], ws0)
            blk2 = blk0 + 2 * NW

            @pl.when(blk2 < NFULL)
            def _():
                pltpu.async_copy(
                    tokT_hbm.at[:, pl.ds(blk2 * TBLK, TBLK)], in0, rs0
                )

        @pl.when(blk1 < NFULL)
        def _():
            pltpu.make_async_copy(
                tokT_hbm.at[:, pl.ds(blk1 * TBLK, TBLK)], in1, rs1
            ).wait()

            @pl.when(j > 0)
            def _():
                drain_w(stg1, ws1)

            _pack_block(in1, stg1, TBLK // LANES)
            pltpu.async_copy(stg1, out_hbm.at[pl.ds(blk1 * PBLK, PBLK)], ws1)

        return carry

    lax.fori_loop(0, APAIRS, pair, 0)
    drain_w(stg0, ws0)
    drain_w(stg1, ws1)

    @pl.when(w == NW - 1)
    def _():
        pltpu.sync_copy(tailT_hbm, tail_v)

        def tb(t, carry):
            r = t >> 1
            cb = (t & 1) * H
            for k in range(H // LANES):
                stg0[r, pl.ds(cb + k * LANES, LANES)] = (
                    tail_v[t, pl.ds(k * LANES, LANES)]
                )
            return carry

        lax.fori_loop(0, TAIL, tb, 0)
        pltpu.sync_copy(
            stg0.at[pl.ds(0, TAIL // 2)],
            out_hbm.at[pl.ds(NFULL * PBLK, TAIL // 2)],
        )


def _compact_add(rows_v, par_v, pos_v, stg, u):
    def grp(g, carry):
        t0 = g * LANES
        tvec = t0 + _i16()
        colb = par_v[u, pl.ds(t0, LANES)] * H
        for f in range(H):
            vec = plsc.load_gather(rows_v, [tvec, colb + f])
            plsc.store_scatter(stg, [tvec, jnp.full((LANES,), f, jnp.int32)], vec)
        return carry

    lax.fori_loop(0, BBLK // LANES, grp, 0)

    p = [pos_v[u, pl.ds(k * LANES, LANES)] for k in range(H // LANES)]

    def rbody(r, carry):
        for k in range(H // LANES):
            s = pl.ds(k * LANES, LANES)
            stg[r, s] = stg[r, s] + p[k]
        return carry

    lax.fori_loop(0, BBLK, rbody, 0, unroll=4)


def _gt_body(xpT_hbm, xrT_hbm, tok_hbm, pos_hbm, out_hbm,
             idx_v, par_v, pos_v, rows0, rows1, stg0, stg1,
             gs0, gs1, ws0, ws1):
    w = lax.axis_index("s") * NC + lax.axis_index("c")
    bblk = lax.rem(w, NBBLK)
    l0 = lax.div(w, NBBLK) * L_PER_W
    cols = pl.ds(bblk * BBLK, BBLK)
    pltpu.sync_copy(xpT_hbm.at[pl.ds(l0, L_PER_W), cols], idx_v)
    pltpu.sync_copy(xrT_hbm.at[pl.ds(l0, L_PER_W), cols], par_v)
    pltpu.sync_copy(pos_hbm.at[pl.ds(l0, L_PER_W)], pos_v)

    def out_slice(u):
        return out_hbm.at[pl.ds(bblk * BBLK, BBLK), pl.ds((l0 + u) * H, H)]

    def drain_w(stg, sem):
        pltpu.make_async_copy(stg, out_slice(0), sem).wait()

    pltpu.async_copy(tok_hbm.at[idx_v.at[0]], rows0, gs0)

    def pair(j, carry):
        a = 2 * j
        b = a + 1
        pltpu.async_copy(tok_hbm.at[idx_v.at[b]], rows1, gs1)
        pltpu.make_async_copy(tok_hbm.at[idx_v.at[a]], rows0, gs0).wait()

        @pl.when(j > 0)
        def _():
            drain_w(stg0, ws0)

        _compact_add(rows0, par_v, pos_v, stg0, a)
        pltpu.async_copy(stg0, out_slice(a), ws0)
        nxt = jnp.minimum(a + 2, UNITS - 1)
        pltpu.async_copy(tok_hbm.at[idx_v.at[nxt]], rows0, gs0)
        pltpu.make_async_copy(tok_hbm.at[idx_v.at[b]], rows1, gs1).wait()

        @pl.when(j > 0)
        def _():
            drain_w(stg1, ws1)

        _compact_add(rows1, par_v, pos_v, stg1, b)
        pltpu.async_copy(stg1, out_slice(b), ws1)
        return carry

    lax.fori_loop(0, UNITS // 2, pair, 0)
    pltpu.make_async_copy(tok_hbm.at[idx_v.at[0]], rows0, gs0).wait()
    drain_w(stg0, ws0)
    drain_w(stg1, ws1)


@functools.cache
def _build_transpose():
    return functools.partial(
        pl.kernel,
        out_type=jax.ShapeDtypeStruct((VP, 2 * H), jnp.float32),
        mesh=plsc.VectorSubcoreMesh(core_axis_name="c", subcore_axis_name="s"),
        scratch_types=[
            pltpu.VMEM((H, TBLK), jnp.float32),
            pltpu.VMEM((H, TBLK), jnp.float32),
            pltpu.VMEM((PBLK, 2 * H), jnp.float32),
            pltpu.VMEM((PBLK, 2 * H), jnp.float32),
            pltpu.VMEM((TAIL, H), jnp.float32),
            pltpu.SemaphoreType.DMA,
            pltpu.SemaphoreType.DMA,
            pltpu.SemaphoreType.DMA,
            pltpu.SemaphoreType.DMA,
        ],
        compiler_params=pltpu.CompilerParams(
            needs_layout_passes=False, disable_bounds_checks=True
        ),
    )(_tr_body)


@functools.cache
def _build_gather():
    return functools.partial(
        pl.kernel,
        out_type=jax.ShapeDtypeStruct((B, L * H), jnp.float32),
        mesh=plsc.VectorSubcoreMesh(core_axis_name="c", subcore_axis_name="s"),
        scratch_types=[
            pltpu.VMEM((L_PER_W, BBLK), jnp.int32),
            pltpu.VMEM((L_PER_W, BBLK), jnp.int32),
            pltpu.VMEM((L_PER_W, H), jnp.float32),
            pltpu.VMEM((BBLK, 2 * H), jnp.float32),
            pltpu.VMEM((BBLK, 2 * H), jnp.float32),
            pltpu.VMEM((BBLK, H), jnp.float32),
            pltpu.VMEM((BBLK, H), jnp.float32),
            pltpu.SemaphoreType.DMA,
            pltpu.SemaphoreType.DMA,
            pltpu.SemaphoreType.DMA,
            pltpu.SemaphoreType.DMA,
        ],
        compiler_params=pltpu.CompilerParams(
            use_tc_tiling_on_sc=False,
            needs_layout_passes=False,
            disable_bounds_checks=True,
        ),
    )(_gt_body)


def kernel(x, token_table, pos_table):
    tailT = token_table[NFULL * TBLK:, :]
    tok_rows = _build_transpose()(token_table.T, tailT)
    xp = (x >> 1).T
    xr = (x & 1).T
    out = _build_gather()(xp, xr, tok_rows, pos_table)
    return out.reshape(B, L, H)


# speedup vs baseline: 2.3113x; 2.2037x over previous
import functools

import jax
import jax.numpy as jnp
from jax import lax
from jax.experimental import pallas as pl
from jax.experimental.pallas import tpu as pltpu
from jax.experimental.pallas import tpu_sc as plsc

B = 1024
L = 200
H = 64
NC = 2
NS = 16
NW = NC * NS
BBLK = 128
NBBLK = B // BBLK
L_PER_W = L * NBBLK // NW
UNITS = L_PER_W
LANES = 16


def _add_stage(pos_v, rows_v, stg, u):
    p = [pos_v[u, pl.ds(k * LANES, LANES)] for k in range(H // LANES)]

    def rbody(r, carry):
        for k in range(H // LANES):
            s = pl.ds(k * LANES, LANES)
            stg[r, s] = rows_v[r, s] + p[k]
        return carry

    lax.fori_loop(0, BBLK, rbody, 0, unroll=4)


def _gt_body(xT_hbm, tok_hbm, pos_hbm, out_hbm,
             idx_v, pos_v, rows0, rows1, stg0, stg1, gs0, gs1, ws0, ws1):
    w = lax.axis_index("s") * NC + lax.axis_index("c")
    bblk = lax.rem(w, NBBLK)
    l0 = lax.div(w, NBBLK) * L_PER_W
    pltpu.sync_copy(
        xT_hbm.at[pl.ds(l0, L_PER_W), pl.ds(bblk * BBLK, BBLK)], idx_v
    )
    pltpu.sync_copy(pos_hbm.at[pl.ds(l0, L_PER_W)], pos_v)

    def out_slice(u):
        return out_hbm.at[pl.ds(bblk * BBLK, BBLK), pl.ds((l0 + u) * H, H)]

    def drain_w(stg, sem):
        pltpu.make_async_copy(stg, out_slice(0), sem).wait()

    pltpu.async_copy(tok_hbm.at[idx_v.at[0]], rows0, gs0)

    def pair(j, carry):
        a = 2 * j
        b = a + 1
        pltpu.async_copy(tok_hbm.at[idx_v.at[b]], rows1, gs1)
        pltpu.make_async_copy(tok_hbm.at[idx_v.at[a]], rows0, gs0).wait()

        @pl.when(j > 0)
        def _():
            drain_w(stg0, ws0)

        _add_stage(pos_v, rows0, stg0, a)
        pltpu.async_copy(stg0, out_slice(a), ws0)
        nxt = jnp.minimum(a + 2, UNITS - 1)
        pltpu.async_copy(tok_hbm.at[idx_v.at[nxt]], rows0, gs0)
        pltpu.make_async_copy(tok_hbm.at[idx_v.at[b]], rows1, gs1).wait()

        @pl.when(j > 0)
        def _():
            drain_w(stg1, ws1)

        _add_stage(pos_v, rows1, stg1, b)
        pltpu.async_copy(stg1, out_slice(b), ws1)
        return carry

    lax.fori_loop(0, UNITS // 2, pair, 0)
    pltpu.make_async_copy(tok_hbm.at[idx_v.at[0]], rows0, gs0).wait()
    drain_w(stg0, ws0)
    drain_w(stg1, ws1)


@functools.cache
def _build_gather():
    return functools.partial(
        pl.kernel,
        out_type=jax.ShapeDtypeStruct((B, L * H), jnp.float32),
        mesh=plsc.VectorSubcoreMesh(core_axis_name="c", subcore_axis_name="s"),
        scratch_types=[
            pltpu.VMEM((L_PER_W, BBLK), jnp.int32),
            pltpu.VMEM((L_PER_W, H), jnp.float32),
            pltpu.VMEM((BBLK, H), jnp.float32),
            pltpu.VMEM((BBLK, H), jnp.float32),
            pltpu.VMEM((BBLK, H), jnp.float32),
            pltpu.VMEM((BBLK, H), jnp.float32),
            pltpu.SemaphoreType.DMA,
            pltpu.SemaphoreType.DMA,
            pltpu.SemaphoreType.DMA,
            pltpu.SemaphoreType.DMA,
        ],
        compiler_params=pltpu.CompilerParams(
            use_tc_tiling_on_sc=False, disable_bounds_checks=True
        ),
    )(_gt_body)


def kernel(x, token_table, pos_table):
    out = _build_gather()(x.T, token_table, pos_table)
    return out.reshape(B, L, H)


# speedup vs baseline: 2.4354x; 1.0537x over previous
import functools

import jax
import jax.numpy as jnp
from jax import lax
from jax.experimental import pallas as pl
from jax.experimental.pallas import tpu as pltpu
from jax.experimental.pallas import tpu_sc as plsc

B = 1024
L = 200
H = 64
NC = 2
NS = 16
NW = NC * NS
BBLK = 128
NBBLK = B // BBLK
L_PER_W = L * NBBLK // NW
UNITS = L_PER_W
LANES = 16


def _add_write(out_hbm, pos_v, rows_v, l0, bblk, u):
    p = [pos_v[u, pl.ds(k * LANES, LANES)] for k in range(H // LANES)]

    def rbody(r, carry):
        for k in range(H // LANES):
            s = pl.ds(k * LANES, LANES)
            rows_v[r, s] = rows_v[r, s] + p[k]
        return carry

    lax.fori_loop(0, BBLK, rbody, 0, unroll=4)
    pltpu.sync_copy(
        rows_v,
        out_hbm.at[pl.ds((l0 + u) * B + bblk * BBLK, BBLK)],
    )


def _emb_body(xT_hbm, tok_hbm, pos_hbm, out_hbm,
              idx_v, pos_v, rows0, rows1, sem0, sem1):
    w = lax.axis_index("s") * NC + lax.axis_index("c")
    bblk = lax.rem(w, NBBLK)
    l0 = lax.div(w, NBBLK) * L_PER_W
    pltpu.sync_copy(
        xT_hbm.at[pl.ds(l0, L_PER_W), pl.ds(bblk * BBLK, BBLK)], idx_v
    )
    pltpu.sync_copy(pos_hbm.at[pl.ds(l0, L_PER_W)], pos_v)

    pltpu.async_copy(tok_hbm.at[idx_v.at[0]], rows0, sem0)

    def pair(j, carry):
        a = 2 * j
        b = a + 1
        pltpu.async_copy(tok_hbm.at[idx_v.at[b]], rows1, sem1)
        pltpu.make_async_copy(tok_hbm.at[idx_v.at[a]], rows0, sem0).wait()
        _add_write(out_hbm, pos_v, rows0, l0, bblk, a)
        nxt = jnp.minimum(a + 2, UNITS - 1)
        pltpu.async_copy(tok_hbm.at[idx_v.at[nxt]], rows0, sem0)
        pltpu.make_async_copy(tok_hbm.at[idx_v.at[b]], rows1, sem1).wait()
        _add_write(out_hbm, pos_v, rows1, l0, bblk, b)
        return carry

    lax.fori_loop(0, UNITS // 2, pair, 0)
    pltpu.make_async_copy(tok_hbm.at[idx_v.at[0]], rows0, sem0).wait()


@functools.cache
def _build_kernel():
    return functools.partial(
        pl.kernel,
        out_type=jax.ShapeDtypeStruct((L * B, H), jnp.float32),
        mesh=plsc.VectorSubcoreMesh(core_axis_name="c", subcore_axis_name="s"),
        scratch_types=[
            pltpu.VMEM((L_PER_W, BBLK), jnp.int32),
            pltpu.VMEM((L_PER_W, H), jnp.float32),
            pltpu.VMEM((BBLK, H), jnp.float32),
            pltpu.VMEM((BBLK, H), jnp.float32),
            pltpu.SemaphoreType.DMA,
            pltpu.SemaphoreType.DMA,
        ],
        compiler_params=pltpu.CompilerParams(use_tc_tiling_on_sc=False),
    )(_emb_body)


def kernel(x, token_table, pos_table):
    out = _build_kernel()(x.T, token_table, pos_table)
    return out.reshape(L, B, H).transpose(1, 0, 2)
